# baseline (speedup 1.0000x reference)
import functools

import jax
import jax.numpy as jnp
from jax import lax
from jax.experimental import pallas as pl
from jax.experimental.pallas import tpu as pltpu
from jax.experimental.pallas import tpu_sc as plsc

N = 10000
E = 320000
G = 64

NW = 32
NPB = 320
NP = NW * NPB
EPT = NP
EPAD = NW * EPT
SINK = 10200
CH = 256
TPAD = 10752
_SC1B = True


def _mesh():
  return plsc.VectorSubcoreMesh(core_axis_name="c", subcore_axis_name="s")


def _wid():
  return lax.axis_index("c") * 16 + lax.axis_index("s")


def _make_segsum(width):
  zrows = NP // 16

  @functools.partial(
      pl.kernel,
      out_type=jax.ShapeDtypeStruct((2, NP, width), jnp.float32),
      mesh=_mesh(),
      compiler_params=pltpu.CompilerParams(use_tc_tiling_on_sc=False,
                                           needs_layout_passes=False),
      scratch_types=[
          pltpu.VMEM((CH,), jnp.int32),
          pltpu.VMEM((CH,), jnp.int32),
          pltpu.VMEM((CH, width), jnp.float32),
          pltpu.VMEM_SHARED((NP, width), jnp.float32),
          pltpu.SemaphoreType.DMA,
      ],
  )
  def k(src_hbm, dst_hbm, table_hbm, zeros_hbm, out_hbm, sidx, didx, rows,
        acc, sem):
    c = lax.axis_index("c")
    s = lax.axis_index("s")
    wid = c * 16 + s
    r0 = s * zrows
    pltpu.sync_copy(zeros_hbm, acc.at[pl.ds(r0, zrows)])
    plsc.subcore_barrier()
    base = wid * EPT

    def step(i, carry):
      off = pl.multiple_of(base + i * CH, 8)
      pltpu.sync_copy(src_hbm.at[pl.ds(off, CH)], sidx)
      pltpu.sync_copy(dst_hbm.at[pl.ds(off, CH)], didx)
      pltpu.async_copy(table_hbm.at[sidx], rows, sem).wait()
      pltpu.sync_copy(rows, acc.at[didx], add=True)
      return carry

    lax.fori_loop(0, EPT // CH, step, 0)
    plsc.subcore_barrier()
    pltpu.sync_copy(acc.at[pl.ds(r0, zrows)], out_hbm.at[c, pl.ds(r0, zrows)])

  return k


_segsum_w144 = _make_segsum(144)
_segsum_w64 = _make_segsum(64)


@functools.partial(
    pl.kernel,
    out_type=(
        jax.ShapeDtypeStruct((NW * TPAD,), jnp.int32),
        jax.ShapeDtypeStruct((NW * TPAD,), jnp.int32),
        jax.ShapeDtypeStruct((NW * NW,), jnp.int32),
        jax.ShapeDtypeStruct((NW * NW,), jnp.int32),
    ),
    mesh=_mesh(),
    compiler_params=pltpu.CompilerParams(needs_layout_passes=False),
    scratch_types=[
        pltpu.VMEM((EPT,), jnp.int32),
        pltpu.VMEM((EPT,), jnp.int32),
        pltpu.VMEM((EPT,), jnp.int32),
        pltpu.VMEM((EPT,), jnp.int32),
        pltpu.VMEM((TPAD,), jnp.int32),
        pltpu.VMEM((TPAD,), jnp.int32),
        pltpu.VMEM((32,), jnp.int32),
        pltpu.VMEM((32,), jnp.int32),
    ],
)
def _sc_bucket(src_hbm, dst_hbm, zeros_hbm, bsrc_o, bdst_o, offs_o, cnts_o,
               srcl, dstl, tsrc, tdst, bsrc_loc, bdst_loc, hist, cur):
  wid = _wid()
  pltpu.sync_copy(src_hbm.at[pl.ds(pl.multiple_of(wid * EPT, 8), EPT)], srcl)
  pltpu.sync_copy(dst_hbm.at[pl.ds(pl.multiple_of(wid * EPT, 8), EPT)], dstl)
  pltpu.sync_copy(zeros_hbm.at[pl.ds(0, TPAD)], bsrc_loc)
  pltpu.sync_copy(zeros_hbm.at[pl.ds(0, TPAD)], bdst_loc)
  zi = jnp.zeros((16,), jnp.int32)
  lanes = lax.iota(jnp.int32, 16)

  def _hist_offsets(align):
    h0 = hist[pl.ds(0, 16)]
    h1 = hist[pl.ds(16, 16)]
    if align:
      h0 = (h0 + 7) & (-8)
      h1 = (h1 + 7) & (-8)
    e0 = plsc.cumsum(h0) - h0
    e1 = plsc.cumsum(h1) - h1 + jnp.sum(h0)
    cur[pl.ds(0, 16)] = e0
    cur[pl.ds(16, 16)] = e1

  def _pass(in_s, in_d, out_s, out_d, keyfn, validfn, nvec, align,
            export_offs=False):
    hist[pl.ds(0, 16)] = zi
    hist[pl.ds(16, 16)] = zi

    def hstep(i, carry):
      d = in_d[pl.ds(i * 16, 16)]
      m = validfn(i, d)
      b = jnp.where(m, keyfn(d), 0)
      r, lastm = plsc.scan_count(b, mask=m)
      plsc.addupdate_scatter(hist, [b], r, mask=lastm)
      return carry

    lax.fori_loop(0, nvec, hstep, 0)
    total = jnp.sum(hist[pl.ds(0, 16)]) + jnp.sum(hist[pl.ds(16, 16)])
    _hist_offsets(align)
    if export_offs:
      pltpu.sync_copy(cur, offs_o.at[pl.ds(pl.multiple_of(wid * 32, 8), 32)])
      pltpu.sync_copy(hist, cnts_o.at[pl.ds(pl.multiple_of(wid * 32, 8), 32)])

    def sstep(i, carry):
      sl = pl.ds(i * 16, 16)
      d = in_d[sl]
      sv = in_s[sl]
      m = validfn(i, d)
      b = jnp.where(m, keyfn(d), 0)
      r, lastm = plsc.scan_count(b, mask=m)
      base = plsc.load_gather(cur, [b], mask=m)
      pos = base + r - 1
      plsc.store_scatter(cur, [b], pos + 1, mask=lastm)
      plsc.store_scatter(out_s, [pos], sv, mask=m)
      plsc.store_scatter(out_d, [pos], d, mask=m)
      return carry

    lax.fori_loop(0, nvec, sstep, 0)
    return total

  cnt = _pass(srcl, dstl, tsrc, tdst,
              lambda d: d & 31,
              lambda i, d: d < N,
              EPT // 16, False)
  nv = (cnt + 15) // 16
  _pass(tsrc, tdst, srcl, dstl,
        lambda d: (d % NPB) >> 5,
        lambda i, d: i * 16 + lanes < cnt,
        nv, False)
  _pass(srcl, dstl, bsrc_loc, bdst_loc,
        lambda d: d // NPB,
        lambda i, d: i * 16 + lanes < cnt,
        nv, True, export_offs=True)
  pltpu.sync_copy(bsrc_loc,
                  bsrc_o.at[pl.ds(pl.multiple_of(wid * TPAD, 8), TPAD)])
  pltpu.sync_copy(bdst_loc,
                  bdst_o.at[pl.ds(pl.multiple_of(wid * TPAD, 8), TPAD)])


MCH = 256


@functools.partial(
    pl.kernel,
    out_type=jax.ShapeDtypeStruct((NP, 128), jnp.float32),
    mesh=_mesh(),
    compiler_params=pltpu.CompilerParams(needs_layout_passes=False),
    scratch_types=[
        pltpu.VMEM((MCH,), jnp.int32),
        pltpu.VMEM((MCH,), jnp.int32),
        pltpu.VMEM((NW * NW,), jnp.int32),
        pltpu.VMEM((NW * NW,), jnp.int32),
        pltpu.VMEM((MCH, 128), jnp.float32),
        pltpu.VMEM((NPB + 8, 128), jnp.float32),
        pltpu.SemaphoreType.DMA,
    ],
)
def _sc_segmax(bsrc, bdst, offs_hbm, cnts_hbm, table_hbm, zacc_hbm, out_hbm,
               sidx, dstv, offs_v, cnts_v, msgs, acc, sem):
  u = _wid()
  pltpu.sync_copy(zacc_hbm, acc)
  pltpu.sync_copy(offs_hbm, offs_v)
  pltpu.sync_copy(cnts_hbm, cnts_v)
  lo = u * NPB
  lanes = lax.iota(jnp.int32, 16)
  uhi = (u // 16) * 16
  ulane = u - uhi
  zv = jnp.zeros((16,), jnp.float32)

  def _flush(dp, regs):
    for j in range(8):
      sl = pl.ds(j * 16, 16)
      acc[dp, sl] = jnp.maximum(acc[dp, sl], regs[j])

  def tloop(t, tcarry):
    ovec = offs_v[pl.ds(pl.multiple_of(t * 32 + uhi, 16), 16)]
    cvec = cnts_v[pl.ds(pl.multiple_of(t * 32 + uhi, 16), 16)]
    off = jnp.sum(jnp.where(lanes == ulane, ovec, 0))
    cnt = jnp.sum(jnp.where(lanes == ulane, cvec, 0))
    nch = (cnt + MCH - 1) // MCH

    def chunk(i, carry):
      stt = pl.multiple_of(t * TPAD + off + i * MCH, 8)
      pltpu.sync_copy(bsrc.at[pl.ds(stt, MCH)], sidx)
      pltpu.sync_copy(bdst.at[pl.ds(stt, MCH)], dstv)
      pltpu.async_copy(table_hbm.at[sidx], msgs, sem).wait()
      rem = cnt - i * MCH

      def grp(gi, gcarry):
        dp = gcarry[0]
        regs = list(gcarry[1:])
        dvec = dstv[pl.ds(gi * 16, 16)] - lo
        for l in range(16):
          e = gi * 16 + l
          d0 = jnp.sum(jnp.where(lanes == l, dvec, 0))
          d = jnp.where(e < rem, d0, NPB)
          changed = d != dp

          @pl.when(changed)
          def _():
            _flush(dp, regs)

          for j in range(8):
            m = msgs[e, pl.ds(j * 16, 16)]
            regs[j] = jnp.where(changed, m, jnp.maximum(regs[j], m))
          dp = d
        return (dp, *regs)

      return lax.fori_loop(0, MCH // 16, grp, carry)

    return lax.fori_loop(0, nch, chunk, tcarry)

  fin = lax.fori_loop(0, NW, tloop,
                      (jnp.int32(NPB), zv, zv, zv, zv, zv, zv, zv, zv))
  _flush(fin[0], list(fin[1:]))
  pltpu.sync_copy(acc.at[pl.ds(0, NPB)],
                  out_hbm.at[pl.ds(pl.multiple_of(lo, 8), NPB)])


@functools.partial(
    pl.kernel,
    out_type=(
        jax.ShapeDtypeStruct((NW, G, 64), jnp.float32),
        jax.ShapeDtypeStruct((NW, G, 64), jnp.float32),
        jax.ShapeDtypeStruct((NW, G, 16), jnp.float32),
    ),
    mesh=_mesh(),
    compiler_params=pltpu.CompilerParams(needs_layout_passes=False),
    scratch_types=[
        pltpu.VMEM((NPB, 64), jnp.float32),
        pltpu.VMEM((NPB,), jnp.int32),
        pltpu.VMEM((G, 64), jnp.float32),
        pltpu.VMEM((G, 64), jnp.float32),
        pltpu.VMEM((G, 16), jnp.float32),
    ],
)
def _sc_pool(h3_hbm, batch_hbm, z64_hbm, z16_hbm, ps_o, pm_o, pc_o,
             rowsb, batchv, acc_s, acc_m, acc_c):
  w = _wid()
  base = pl.multiple_of(w * NPB, 8)
  pltpu.sync_copy(h3_hbm.at[pl.ds(base, NPB)], rowsb)
  pltpu.sync_copy(batch_hbm.at[pl.ds(base, NPB)], batchv)
  pltpu.sync_copy(z64_hbm, acc_s)
  pltpu.sync_copy(z64_hbm, acc_m)
  pltpu.sync_copy(z16_hbm, acc_c)
  lanes = lax.iota(jnp.int32, 16)
  onev = jnp.where(lanes == 0, 1.0, 0.0).astype(jnp.float32)

  def grp(gi, carry):
    bvec = batchv[pl.ds(gi * 16, 16)]
    for l in range(16):
      r = gi * 16 + l

      @pl.when(w * NPB + r < N)
      def _():
        g = jnp.sum(jnp.where(lanes == l, bvec, 0))
        for j in range(4):
          sl = pl.ds(j * 16, 16)
          v = rowsb[r, sl]
          acc_s[g, sl] = acc_s[g, sl] + v
          acc_m[g, sl] = jnp.maximum(acc_m[g, sl], v)
        c16 = pl.ds(0, 16)
        acc_c[g, c16] = acc_c[g, c16] + onev

    return carry

  lax.fori_loop(0, NPB // 16, grp, 0)
  pltpu.sync_copy(acc_s, ps_o.at[w])
  pltpu.sync_copy(acc_m, pm_o.at[w])
  pltpu.sync_copy(acc_c, pc_o.at[w])


_RB = 2000


def _mm_body(x_ref, w_ref, o_ref):
  o_ref[...] = jnp.dot(x_ref[...], w_ref[...],
                       preferred_element_type=jnp.float32)


def _tc_mm(x, w):
  n, kdim = x.shape
  m = w.shape[1]
  return pl.pallas_call(
      _mm_body,
      grid=(n // _RB,),
      in_specs=[
          pl.BlockSpec((_RB, kdim), lambda i: (i, 0)),
          pl.BlockSpec((kdim, m), lambda i: (0, 0)),
      ],
      out_specs=pl.BlockSpec((_RB, m), lambda i: (i, 0)),
      out_shape=jax.ShapeDtypeStruct((n, m), jnp.float32),
  )(x, w)


def _layer_mean_body(s0_ref, s1_ref, d0_ref, d1_ref, x_ref, w_ref, b_ref,
                     o_ref):
  s = s0_ref[...] + s1_ref[...]
  deg = jnp.sum(d0_ref[...] + d1_ref[...], axis=1, keepdims=True)
  agg = s / jnp.maximum(deg, 1.0)
  o_ref[...] = jax.nn.relu(agg + b_ref[...] +
                           jnp.dot(x_ref[...], w_ref[...],
                                   preferred_element_type=jnp.float32))


def _tc_layer_mean(s0, s1, d0, d1, x, w, b):
  n, kdim = x.shape
  m = w.shape[1]
  return pl.pallas_call(
      _layer_mean_body,
      grid=(n // _RB,),
      in_specs=[
          pl.BlockSpec((_RB, m), lambda i: (i, 0)),
          pl.BlockSpec((_RB, m), lambda i: (i, 0)),
          pl.BlockSpec((_RB, 16), lambda i: (i, 0)),
          pl.BlockSpec((_RB, 16), lambda i: (i, 0)),
          pl.BlockSpec((_RB, kdim), lambda i: (i, 0)),
          pl.BlockSpec((kdim, m), lambda i: (0, 0)),
          pl.BlockSpec((1, m), lambda i: (0, 0)),
      ],
      out_specs=pl.BlockSpec((_RB, m), lambda i: (i, 0)),
      out_shape=jax.ShapeDtypeStruct((n, m), jnp.float32),
  )(s0, s1, d0, d1, x, w, b)


def _layer2_body(m_ref, h1_ref, wl_ref, wr_ref, b_ref, w3_ref, h2_ref,
                 p3_ref):
  h2 = jax.nn.relu(
      jnp.dot(m_ref[...], wl_ref[...], preferred_element_type=jnp.float32)
      + b_ref[...]
      + jnp.dot(h1_ref[...], wr_ref[...], preferred_element_type=jnp.float32))
  h2_ref[...] = h2
  p3_ref[...] = jnp.dot(h2, w3_ref[...], preferred_element_type=jnp.float32)


def _tc_layer2(m, h1, wl, wr, b, w3):
  n = m.shape[0]
  return pl.pallas_call(
      _layer2_body,
      grid=(n // _RB,),
      in_specs=[
          pl.BlockSpec((_RB, 128), lambda i: (i, 0)),
          pl.BlockSpec((_RB, 128), lambda i: (i, 0)),
          pl.BlockSpec((128, 128), lambda i: (0, 0)),
          pl.BlockSpec((128, 128), lambda i: (0, 0)),
          pl.BlockSpec((1, 128), lambda i: (0, 0)),
          pl.BlockSpec((128, 64), lambda i: (0, 0)),
      ],
      out_specs=(
          pl.BlockSpec((_RB, 128), lambda i: (i, 0)),
          pl.BlockSpec((_RB, 64), lambda i: (i, 0)),
      ),
      out_shape=(
          jax.ShapeDtypeStruct((n, 128), jnp.float32),
          jax.ShapeDtypeStruct((n, 64), jnp.float32),
      ),
  )(m, h1, wl, wr, b, w3)


def _gelu(z):
  return 0.5 * z * (1.0 + lax.erf(z * 0.7071067811865476))


def _lnorm(p, g, b):
  m = jnp.mean(p, axis=-1, keepdims=True)
  r = p - m
  v = jnp.mean(r * r, axis=-1, keepdims=True)
  return r * lax.rsqrt(v + 1e-5) * g + b


def _head_body(ps_ref, pm_ref, pc_ref, w1_ref, b1_ref, w2_ref, b2_ref,
               w3_ref, b3_ref, w4_ref, b4_ref, w5_ref, b5_ref, g1_ref,
               be1_ref, g2_ref, be2_ref, g3_ref, be3_ref, g4_ref, be4_ref,
               o_ref):
  ps = jnp.zeros((G, 64), jnp.float32)
  pm = jnp.zeros((G, 64), jnp.float32)
  pc = jnp.zeros((G, 16), jnp.float32)
  for t in range(NW):
    ps = ps + ps_ref[t]
    pm = jnp.maximum(pm, pm_ref[t])
    pc = pc + pc_ref[t]
  cnt = pc[:, 0:1]
  mean_p = ps / jnp.maximum(cnt, 1.0)
  p = jnp.concatenate([mean_p, pm, ps], axis=-1)
  p = _lnorm(_gelu(jnp.dot(p, w1_ref[...],
                           preferred_element_type=jnp.float32) + b1_ref[...]),
             g1_ref[...], be1_ref[...])
  p = _lnorm(_gelu(jnp.dot(p, w2_ref[...],
                           preferred_element_type=jnp.float32) + b2_ref[...]),
             g2_ref[...], be2_ref[...])
  p = _lnorm(_gelu(jnp.dot(p, w3_ref[...],
                           preferred_element_type=jnp.float32) + b3_ref[...]),
             g3_ref[...], be3_ref[...])
  p = _lnorm(_gelu(jnp.dot(p, w4_ref[...],
                           preferred_element_type=jnp.float32) + b4_ref[...]),
             g4_ref[...], be4_ref[...])
  o_ref[...] = jnp.dot(p, w5_ref[...],
                       preferred_element_type=jnp.float32) + b5_ref[...]


def _tc_head(ps, pm, pc, hw, hb, gs, bes):
  full = lambda s: pl.BlockSpec(s, lambda: tuple(0 for _ in s))
  in_specs = [full((NW, G, 64)), full((NW, G, 64)), full((NW, G, 16))]
  args = [ps, pm, pc]
  for i in range(5):
    in_specs += [full(hw[i].shape), full((1, hw[i].shape[1]))]
    args += [hw[i], hb[i].reshape(1, -1)]
  for i in range(4):
    in_specs += [full((1, gs[i].shape[0])), full((1, gs[i].shape[0]))]
    args += [gs[i].reshape(1, -1), bes[i].reshape(1, -1)]
  return pl.pallas_call(
      _head_body,
      in_specs=in_specs,
      out_specs=full((G, 9)),
      out_shape=jax.ShapeDtypeStruct((G, 9), jnp.float32),
  )(*args)


def kernel(x, edge_index, batch, W1l, b1, W1r, W2l, b2, W2r, W3l, b3, W3r,
           hW1, hb1, hW2, hb2, hW3, hb3, hW4, hb4, hW5, hb5, g1, be1, g2,
           be2, g3, be3, g4, be4):
  f32 = jnp.float32
  src = edge_index[0]
  dst = edge_index[1]
  npad = EPAD - E
  park = jnp.arange(npad, dtype=jnp.int32)
  src_p = jnp.concatenate([src, park % N])
  dst_p = jnp.concatenate([dst, N + park % (NP - N)])

  zeros_w144 = jnp.zeros((NP // 16, 144), f32)
  zeros_w64 = jnp.zeros((NP // 16, 64), f32)
  zeros_i32 = jnp.zeros((TPAD,), jnp.int32)
  zacc = jnp.zeros((NPB + 8, 128), f32)
  z64 = jnp.zeros((G, 64), f32)
  z16 = jnp.zeros((G, 16), f32)

  p1 = _tc_mm(x, W1l)
  p1aug = jnp.concatenate(
      [p1, jnp.ones((N, 1), f32), jnp.zeros((N, 15), f32)], axis=1)
  s1 = _segsum_w144(src_p, dst_p, p1aug, zeros_w144)
  s1a = s1[0, :, :128]
  s1b = s1[1, :, :128]
  d1a = s1[0, :, 128:144]
  d1b = s1[1, :, 128:144]
  h1 = _tc_layer_mean(s1a, s1b, d1a, d1b, x, W1r, b1.reshape(1, -1))

  bsrc, bdst, offs, cnts = _sc_bucket(src_p, dst_p, zeros_i32)

  m = _sc_segmax(bsrc, bdst, offs, cnts, h1, zacc)
  h2, p3 = _tc_layer2(m[:N], h1, W2l, W2r, b2.reshape(1, -1), W3l)

  s3 = _segsum_w64(src_p, dst_p, p3, zeros_w64)
  h3 = _tc_layer_mean(s3[0], s3[1], d1a, d1b, h2, W3r, b3.reshape(1, -1))

  h3p = jnp.concatenate([h3, jnp.zeros((NP - N, 64), f32)])
  batch_p = jnp.concatenate([batch, jnp.zeros((NP - N,), jnp.int32)])
  ps, pm, pc = _sc_pool(h3p, batch_p, z64, z16)
  return _tc_head(ps, pm, pc, (hW1, hW2, hW3, hW4, hW5),
                  (hb1, hb2, hb3, hb4, hb5), (g1, g2, g3, g4),
                  (be1, be2, be3, be4))


# speedup vs baseline: 1.4262x; 1.4262x over previous
import functools

import jax
import jax.numpy as jnp
from jax import lax
from jax.experimental import pallas as pl
from jax.experimental.pallas import tpu as pltpu
from jax.experimental.pallas import tpu_sc as plsc

N = 10000
E = 320000
G = 64

NW = 32
NPB = 320
NP = NW * NPB
EPT = NP
EPAD = NW * EPT
SINK = 10200
CH = 128
TPAD = 10752
_SC1B = True


def _mesh():
  return plsc.VectorSubcoreMesh(core_axis_name="c", subcore_axis_name="s")


def _wid():
  return lax.axis_index("c") * 16 + lax.axis_index("s")


def _make_segsum(width):
  zrows = NP // 16

  @functools.partial(
      pl.kernel,
      out_type=jax.ShapeDtypeStruct((2, NP, width), jnp.float32),
      mesh=_mesh(),
      compiler_params=pltpu.CompilerParams(use_tc_tiling_on_sc=False,
                                           needs_layout_passes=False),
      scratch_types=[
          pltpu.VMEM((CH,), jnp.int32),
          pltpu.VMEM((CH,), jnp.int32),
          pltpu.VMEM((CH, width), jnp.float32),
          pltpu.VMEM_SHARED((NP, width), jnp.float32),
          pltpu.SemaphoreType.DMA,
          pltpu.SemaphoreType.DMA,
          pltpu.SemaphoreType.DMA,
      ],
  )
  def k(src_hbm, dst_hbm, table_hbm, zeros_hbm, out_hbm, sidx, didx, rows,
        acc, sem, semA, semB):
    c = lax.axis_index("c")
    s = lax.axis_index("s")
    wid = c * 16 + s
    r0 = s * zrows
    pltpu.sync_copy(zeros_hbm, acc.at[pl.ds(r0, zrows)])
    plsc.subcore_barrier()
    base = wid * EPT

    def step(i, carry):
      off = pl.multiple_of(base + i * CH, 8)
      cs = pltpu.async_copy(src_hbm.at[pl.ds(off, CH)], sidx, semA)
      cd = pltpu.async_copy(dst_hbm.at[pl.ds(off, CH)], didx, semB)
      cs.wait()
      pltpu.async_copy(table_hbm.at[sidx], rows, sem).wait()
      cd.wait()
      pltpu.sync_copy(rows, acc.at[didx], add=True)
      return carry

    lax.fori_loop(0, EPT // CH, step, 0)
    plsc.subcore_barrier()
    pltpu.sync_copy(acc.at[pl.ds(r0, zrows)], out_hbm.at[c, pl.ds(r0, zrows)])

  return k


_segsum_w144 = _make_segsum(144)
_segsum_w64 = _make_segsum(64)


@functools.partial(
    pl.kernel,
    out_type=(
        jax.ShapeDtypeStruct((NW * TPAD,), jnp.int32),
        jax.ShapeDtypeStruct((NW * TPAD,), jnp.int32),
        jax.ShapeDtypeStruct((NW * NW,), jnp.int32),
        jax.ShapeDtypeStruct((NW * NW,), jnp.int32),
    ),
    mesh=_mesh(),
    compiler_params=pltpu.CompilerParams(needs_layout_passes=False),
    scratch_types=[
        pltpu.VMEM((EPT,), jnp.int32),
        pltpu.VMEM((EPT,), jnp.int32),
        pltpu.VMEM((EPT,), jnp.int32),
        pltpu.VMEM((EPT,), jnp.int32),
        pltpu.VMEM((TPAD,), jnp.int32),
        pltpu.VMEM((TPAD,), jnp.int32),
        pltpu.VMEM((32,), jnp.int32),
        pltpu.VMEM((32,), jnp.int32),
    ],
)
def _sc_bucket(src_hbm, dst_hbm, zeros_hbm, bsrc_o, bdst_o, offs_o, cnts_o,
               srcl, dstl, tsrc, tdst, bsrc_loc, bdst_loc, hist, cur):
  wid = _wid()
  pltpu.sync_copy(src_hbm.at[pl.ds(pl.multiple_of(wid * EPT, 8), EPT)], srcl)
  pltpu.sync_copy(dst_hbm.at[pl.ds(pl.multiple_of(wid * EPT, 8), EPT)], dstl)
  pltpu.sync_copy(zeros_hbm.at[pl.ds(0, TPAD)], bsrc_loc)
  pltpu.sync_copy(zeros_hbm.at[pl.ds(0, TPAD)], bdst_loc)
  zi = jnp.zeros((16,), jnp.int32)
  lanes = lax.iota(jnp.int32, 16)

  def _hist_offsets(align):
    h0 = hist[pl.ds(0, 16)]
    h1 = hist[pl.ds(16, 16)]
    if align:
      h0 = (h0 + 7) & (-8)
      h1 = (h1 + 7) & (-8)
    e0 = plsc.cumsum(h0) - h0
    e1 = plsc.cumsum(h1) - h1 + jnp.sum(h0)
    cur[pl.ds(0, 16)] = e0
    cur[pl.ds(16, 16)] = e1

  def _pass(in_s, in_d, out_s, out_d, keyfn, validfn, nvec, align,
            export_offs=False):
    hist[pl.ds(0, 16)] = zi
    hist[pl.ds(16, 16)] = zi

    def hstep(i, carry):
      d = in_d[pl.ds(i * 16, 16)]
      m = validfn(i, d)
      b = jnp.where(m, keyfn(d), 0)
      r, lastm = plsc.scan_count(b, mask=m)
      plsc.addupdate_scatter(hist, [b], r, mask=lastm)
      return carry

    lax.fori_loop(0, nvec, hstep, 0)
    total = jnp.sum(hist[pl.ds(0, 16)]) + jnp.sum(hist[pl.ds(16, 16)])
    _hist_offsets(align)
    if export_offs:
      pltpu.sync_copy(cur, offs_o.at[pl.ds(pl.multiple_of(wid * 32, 8), 32)])
      pltpu.sync_copy(hist, cnts_o.at[pl.ds(pl.multiple_of(wid * 32, 8), 32)])

    def sstep(i, carry):
      sl = pl.ds(i * 16, 16)
      d = in_d[sl]
      sv = in_s[sl]
      m = validfn(i, d)
      b = jnp.where(m, keyfn(d), 0)
      r, lastm = plsc.scan_count(b, mask=m)
      base = plsc.load_gather(cur, [b], mask=m)
      pos = base + r - 1
      plsc.store_scatter(cur, [b], pos + 1, mask=lastm)
      plsc.store_scatter(out_s, [pos], sv, mask=m)
      plsc.store_scatter(out_d, [pos], d, mask=m)
      return carry

    lax.fori_loop(0, nvec, sstep, 0)
    return total

  cnt = _pass(srcl, dstl, tsrc, tdst,
              lambda d: d & 31,
              lambda i, d: d < N,
              EPT // 16, False)
  nv = (cnt + 15) // 16
  _pass(tsrc, tdst, srcl, dstl,
        lambda d: (d % NPB) >> 5,
        lambda i, d: i * 16 + lanes < cnt,
        nv, False)
  _pass(srcl, dstl, bsrc_loc, bdst_loc,
        lambda d: d // NPB,
        lambda i, d: i * 16 + lanes < cnt,
        nv, True, export_offs=True)
  pltpu.sync_copy(bsrc_loc,
                  bsrc_o.at[pl.ds(pl.multiple_of(wid * TPAD, 8), TPAD)])
  pltpu.sync_copy(bdst_loc,
                  bdst_o.at[pl.ds(pl.multiple_of(wid * TPAD, 8), TPAD)])


MCH = 128


@functools.partial(
    pl.kernel,
    out_type=jax.ShapeDtypeStruct((NP, 128), jnp.float32),
    mesh=_mesh(),
    compiler_params=pltpu.CompilerParams(needs_layout_passes=False),
    scratch_types=[
        pltpu.VMEM((MCH,), jnp.int32),
        pltpu.VMEM((MCH,), jnp.int32),
        pltpu.VMEM((NW * NW,), jnp.int32),
        pltpu.VMEM((NW * NW,), jnp.int32),
        pltpu.VMEM((MCH, 128), jnp.float32),
        pltpu.VMEM((NPB + 8, 128), jnp.float32),
        pltpu.SemaphoreType.DMA,
        pltpu.SemaphoreType.DMA,
        pltpu.SemaphoreType.DMA,
    ],
)
def _sc_segmax(bsrc, bdst, offs_hbm, cnts_hbm, table_hbm, zacc_hbm, out_hbm,
               sidx, dstv, offs_v, cnts_v, msgs, acc, sem, semA, semB):
  u = _wid()
  pltpu.sync_copy(zacc_hbm, acc)
  pltpu.sync_copy(offs_hbm, offs_v)
  pltpu.sync_copy(cnts_hbm, cnts_v)
  lo = u * NPB
  lanes = lax.iota(jnp.int32, 16)
  uhi = (u // 16) * 16
  ulane = u - uhi
  zv = jnp.zeros((16,), jnp.float32)

  def _flush(dp, regs):
    for j in range(8):
      sl = pl.ds(j * 16, 16)
      acc[dp, sl] = jnp.maximum(acc[dp, sl], regs[j])

  def tloop(t, tcarry):
    ovec = offs_v[pl.ds(pl.multiple_of(t * 32 + uhi, 16), 16)]
    cvec = cnts_v[pl.ds(pl.multiple_of(t * 32 + uhi, 16), 16)]
    off = jnp.sum(jnp.where(lanes == ulane, ovec, 0))
    cnt = jnp.sum(jnp.where(lanes == ulane, cvec, 0))
    nch = (cnt + MCH - 1) // MCH

    def chunk(i, carry):
      stt = pl.multiple_of(t * TPAD + off + i * MCH, 8)
      cs = pltpu.async_copy(bsrc.at[pl.ds(stt, MCH)], sidx, semA)
      cd = pltpu.async_copy(bdst.at[pl.ds(stt, MCH)], dstv, semB)
      cs.wait()
      g = pltpu.async_copy(table_hbm.at[sidx], msgs, sem)
      cd.wait()
      g.wait()
      rem = cnt - i * MCH

      def grp(gi, gcarry):
        dp = gcarry[0]
        regs = list(gcarry[1:])
        dvec = dstv[pl.ds(gi * 16, 16)] - lo
        for l in range(16):
          e = gi * 16 + l
          d0 = jnp.sum(jnp.where(lanes == l, dvec, 0))
          d = jnp.where(e < rem, d0, NPB)
          changed = d != dp

          @pl.when(changed)
          def _():
            _flush(dp, regs)

          for j in range(8):
            m = msgs[e, pl.ds(j * 16, 16)]
            regs[j] = jnp.where(changed, m, jnp.maximum(regs[j], m))
          dp = d
        return (dp, *regs)

      return lax.fori_loop(0, MCH // 16, grp, carry)

    return lax.fori_loop(0, nch, chunk, tcarry)

  fin = lax.fori_loop(0, NW, tloop,
                      (jnp.int32(NPB), zv, zv, zv, zv, zv, zv, zv, zv))
  _flush(fin[0], list(fin[1:]))
  pltpu.sync_copy(acc.at[pl.ds(0, NPB)],
                  out_hbm.at[pl.ds(pl.multiple_of(lo, 8), NPB)])


@functools.partial(
    pl.kernel,
    out_type=(
        jax.ShapeDtypeStruct((NW, G, 64), jnp.float32),
        jax.ShapeDtypeStruct((NW, G, 64), jnp.float32),
        jax.ShapeDtypeStruct((NW, G, 16), jnp.float32),
    ),
    mesh=_mesh(),
    compiler_params=pltpu.CompilerParams(needs_layout_passes=False),
    scratch_types=[
        pltpu.VMEM((NPB, 64), jnp.float32),
        pltpu.VMEM((NPB,), jnp.int32),
        pltpu.VMEM((G, 64), jnp.float32),
        pltpu.VMEM((G, 64), jnp.float32),
        pltpu.VMEM((G, 16), jnp.float32),
    ],
)
def _sc_pool(h3_hbm, batch_hbm, z64_hbm, z16_hbm, ps_o, pm_o, pc_o,
             rowsb, batchv, acc_s, acc_m, acc_c):
  w = _wid()
  base = pl.multiple_of(w * NPB, 8)
  pltpu.sync_copy(h3_hbm.at[pl.ds(base, NPB)], rowsb)
  pltpu.sync_copy(batch_hbm.at[pl.ds(base, NPB)], batchv)
  pltpu.sync_copy(z64_hbm, acc_s)
  pltpu.sync_copy(z64_hbm, acc_m)
  pltpu.sync_copy(z16_hbm, acc_c)
  lanes = lax.iota(jnp.int32, 16)
  onev = jnp.where(lanes == 0, 1.0, 0.0).astype(jnp.float32)

  def grp(gi, carry):
    bvec = batchv[pl.ds(gi * 16, 16)]
    for l in range(16):
      r = gi * 16 + l

      @pl.when(w * NPB + r < N)
      def _():
        g = jnp.sum(jnp.where(lanes == l, bvec, 0))
        for j in range(4):
          sl = pl.ds(j * 16, 16)
          v = rowsb[r, sl]
          acc_s[g, sl] = acc_s[g, sl] + v
          acc_m[g, sl] = jnp.maximum(acc_m[g, sl], v)
        c16 = pl.ds(0, 16)
        acc_c[g, c16] = acc_c[g, c16] + onev

    return carry

  lax.fori_loop(0, NPB // 16, grp, 0)
  pltpu.sync_copy(acc_s, ps_o.at[w])
  pltpu.sync_copy(acc_m, pm_o.at[w])
  pltpu.sync_copy(acc_c, pc_o.at[w])


_RB = 2000


def _mm_body(x_ref, w_ref, o_ref):
  o_ref[...] = jnp.dot(x_ref[...], w_ref[...],
                       preferred_element_type=jnp.float32)


def _tc_mm(x, w):
  n, kdim = x.shape
  m = w.shape[1]
  return pl.pallas_call(
      _mm_body,
      grid=(n // _RB,),
      in_specs=[
          pl.BlockSpec((_RB, kdim), lambda i: (i, 0)),
          pl.BlockSpec((kdim, m), lambda i: (0, 0)),
      ],
      out_specs=pl.BlockSpec((_RB, m), lambda i: (i, 0)),
      out_shape=jax.ShapeDtypeStruct((n, m), jnp.float32),
  )(x, w)


def _layer_mean_body(s0_ref, s1_ref, d0_ref, d1_ref, x_ref, w_ref, b_ref,
                     o_ref):
  s = s0_ref[...] + s1_ref[...]
  deg = jnp.sum(d0_ref[...] + d1_ref[...], axis=1, keepdims=True)
  agg = s / jnp.maximum(deg, 1.0)
  o_ref[...] = jax.nn.relu(agg + b_ref[...] +
                           jnp.dot(x_ref[...], w_ref[...],
                                   preferred_element_type=jnp.float32))


def _tc_layer_mean(s0, s1, d0, d1, x, w, b):
  n, kdim = x.shape
  m = w.shape[1]
  return pl.pallas_call(
      _layer_mean_body,
      grid=(n // _RB,),
      in_specs=[
          pl.BlockSpec((_RB, m), lambda i: (i, 0)),
          pl.BlockSpec((_RB, m), lambda i: (i, 0)),
          pl.BlockSpec((_RB, 16), lambda i: (i, 0)),
          pl.BlockSpec((_RB, 16), lambda i: (i, 0)),
          pl.BlockSpec((_RB, kdim), lambda i: (i, 0)),
          pl.BlockSpec((kdim, m), lambda i: (0, 0)),
          pl.BlockSpec((1, m), lambda i: (0, 0)),
      ],
      out_specs=pl.BlockSpec((_RB, m), lambda i: (i, 0)),
      out_shape=jax.ShapeDtypeStruct((n, m), jnp.float32),
  )(s0, s1, d0, d1, x, w, b)


def _layer2_body(m_ref, h1_ref, wl_ref, wr_ref, b_ref, w3_ref, h2_ref,
                 p3_ref):
  h2 = jax.nn.relu(
      jnp.dot(m_ref[...], wl_ref[...], preferred_element_type=jnp.float32)
      + b_ref[...]
      + jnp.dot(h1_ref[...], wr_ref[...], preferred_element_type=jnp.float32))
  h2_ref[...] = h2
  p3_ref[...] = jnp.dot(h2, w3_ref[...], preferred_element_type=jnp.float32)


def _tc_layer2(m, h1, wl, wr, b, w3):
  n = m.shape[0]
  return pl.pallas_call(
      _layer2_body,
      grid=(n // _RB,),
      in_specs=[
          pl.BlockSpec((_RB, 128), lambda i: (i, 0)),
          pl.BlockSpec((_RB, 128), lambda i: (i, 0)),
          pl.BlockSpec((128, 128), lambda i: (0, 0)),
          pl.BlockSpec((128, 128), lambda i: (0, 0)),
          pl.BlockSpec((1, 128), lambda i: (0, 0)),
          pl.BlockSpec((128, 64), lambda i: (0, 0)),
      ],
      out_specs=(
          pl.BlockSpec((_RB, 128), lambda i: (i, 0)),
          pl.BlockSpec((_RB, 64), lambda i: (i, 0)),
      ),
      out_shape=(
          jax.ShapeDtypeStruct((n, 128), jnp.float32),
          jax.ShapeDtypeStruct((n, 64), jnp.float32),
      ),
  )(m, h1, wl, wr, b, w3)


def _gelu(z):
  return 0.5 * z * (1.0 + lax.erf(z * 0.7071067811865476))


def _lnorm(p, g, b):
  m = jnp.mean(p, axis=-1, keepdims=True)
  r = p - m
  v = jnp.mean(r * r, axis=-1, keepdims=True)
  return r * lax.rsqrt(v + 1e-5) * g + b


def _head_body(ps_ref, pm_ref, pc_ref, w1_ref, b1_ref, w2_ref, b2_ref,
               w3_ref, b3_ref, w4_ref, b4_ref, w5_ref, b5_ref, g1_ref,
               be1_ref, g2_ref, be2_ref, g3_ref, be3_ref, g4_ref, be4_ref,
               o_ref):
  ps = jnp.zeros((G, 64), jnp.float32)
  pm = jnp.zeros((G, 64), jnp.float32)
  pc = jnp.zeros((G, 16), jnp.float32)
  for t in range(NW):
    ps = ps + ps_ref[t]
    pm = jnp.maximum(pm, pm_ref[t])
    pc = pc + pc_ref[t]
  cnt = pc[:, 0:1]
  mean_p = ps / jnp.maximum(cnt, 1.0)
  p = jnp.concatenate([mean_p, pm, ps], axis=-1)
  p = _lnorm(_gelu(jnp.dot(p, w1_ref[...],
                           preferred_element_type=jnp.float32) + b1_ref[...]),
             g1_ref[...], be1_ref[...])
  p = _lnorm(_gelu(jnp.dot(p, w2_ref[...],
                           preferred_element_type=jnp.float32) + b2_ref[...]),
             g2_ref[...], be2_ref[...])
  p = _lnorm(_gelu(jnp.dot(p, w3_ref[...],
                           preferred_element_type=jnp.float32) + b3_ref[...]),
             g3_ref[...], be3_ref[...])
  p = _lnorm(_gelu(jnp.dot(p, w4_ref[...],
                           preferred_element_type=jnp.float32) + b4_ref[...]),
             g4_ref[...], be4_ref[...])
  o_ref[...] = jnp.dot(p, w5_ref[...],
                       preferred_element_type=jnp.float32) + b5_ref[...]


def _tc_head(ps, pm, pc, hw, hb, gs, bes):
  full = lambda s: pl.BlockSpec(s, lambda: tuple(0 for _ in s))
  in_specs = [full((NW, G, 64)), full((NW, G, 64)), full((NW, G, 16))]
  args = [ps, pm, pc]
  for i in range(5):
    in_specs += [full(hw[i].shape), full((1, hw[i].shape[1]))]
    args += [hw[i], hb[i].reshape(1, -1)]
  for i in range(4):
    in_specs += [full((1, gs[i].shape[0])), full((1, gs[i].shape[0]))]
    args += [gs[i].reshape(1, -1), bes[i].reshape(1, -1)]
  return pl.pallas_call(
      _head_body,
      in_specs=in_specs,
      out_specs=full((G, 9)),
      out_shape=jax.ShapeDtypeStruct((G, 9), jnp.float32),
  )(*args)


def kernel(x, edge_index, batch, W1l, b1, W1r, W2l, b2, W2r, W3l, b3, W3r,
           hW1, hb1, hW2, hb2, hW3, hb3, hW4, hb4, hW5, hb5, g1, be1, g2,
           be2, g3, be3, g4, be4):
  f32 = jnp.float32
  src = edge_index[0]
  dst = edge_index[1]
  npad = EPAD - E
  park = jnp.arange(npad, dtype=jnp.int32)
  src_p = jnp.concatenate([src, park % N])
  dst_p = jnp.concatenate([dst, N + park % (NP - N)])

  zeros_w144 = jnp.zeros((NP // 16, 144), f32)
  zeros_w64 = jnp.zeros((NP // 16, 64), f32)
  zeros_i32 = jnp.zeros((TPAD,), jnp.int32)
  zacc = jnp.zeros((NPB + 8, 128), f32)
  z64 = jnp.zeros((G, 64), f32)
  z16 = jnp.zeros((G, 16), f32)

  p1 = _tc_mm(x, W1l)
  p1aug = jnp.concatenate(
      [p1, jnp.ones((N, 1), f32), jnp.zeros((N, 15), f32)], axis=1)
  s1 = _segsum_w144(src_p, dst_p, p1aug, zeros_w144)
  s1a = s1[0, :, :128]
  s1b = s1[1, :, :128]
  d1a = s1[0, :, 128:144]
  d1b = s1[1, :, 128:144]
  h1 = _tc_layer_mean(s1a, s1b, d1a, d1b, x, W1r, b1.reshape(1, -1))

  bsrc, bdst, offs, cnts = _sc_bucket(src_p, dst_p, zeros_i32)

  m = _sc_segmax(bsrc, bdst, offs, cnts, h1, zacc)
  h2, p3 = _tc_layer2(m[:N], h1, W2l, W2r, b2.reshape(1, -1), W3l)

  s3 = _segsum_w64(src_p, dst_p, p3, zeros_w64)
  h3 = _tc_layer_mean(s3[0], s3[1], d1a, d1b, h2, W3r, b3.reshape(1, -1))

  h3p = jnp.concatenate([h3, jnp.zeros((NP - N, 64), f32)])
  batch_p = jnp.concatenate([batch, jnp.zeros((NP - N,), jnp.int32)])
  ps, pm, pc = _sc_pool(h3p, batch_p, z64, z16)
  return _tc_head(ps, pm, pc, (hW1, hW2, hW3, hW4, hW5),
                  (hb1, hb2, hb3, hb4, hb5), (g1, g2, g3, g4),
                  (be1, be2, be3, be4))


# speedup vs baseline: 1.4275x; 1.0009x over previous
import functools

import jax
import jax.numpy as jnp
from jax import lax
from jax.experimental import pallas as pl
from jax.experimental.pallas import tpu as pltpu
from jax.experimental.pallas import tpu_sc as plsc

N = 10000
E = 320000
G = 64

NW = 32
NPB = 320
NP = NW * NPB
EPT = NP
EPAD = NW * EPT
CH = 128
TPAD = 10752
_SC1B = True


def _mesh():
  return plsc.VectorSubcoreMesh(core_axis_name="c", subcore_axis_name="s")


def _wid():
  return lax.axis_index("c") * 16 + lax.axis_index("s")


def _make_segsum(width):
  zrows = NP // 16

  @functools.partial(
      pl.kernel,
      out_type=jax.ShapeDtypeStruct((2, NP, width), jnp.float32),
      mesh=_mesh(),
      compiler_params=pltpu.CompilerParams(use_tc_tiling_on_sc=False,
                                           needs_layout_passes=False),
      scratch_types=[
          pltpu.VMEM((CH,), jnp.int32),
          pltpu.VMEM((CH,), jnp.int32),
          pltpu.VMEM((CH, width), jnp.float32),
          pltpu.VMEM_SHARED((NP, width), jnp.float32),
          pltpu.SemaphoreType.DMA,
          pltpu.SemaphoreType.DMA,
          pltpu.SemaphoreType.DMA,
      ],
  )
  def k(src_hbm, dst_hbm, table_hbm, zeros_hbm, out_hbm, sidx, didx, rows,
        acc, sem, semA, semB):
    c = lax.axis_index("c")
    s = lax.axis_index("s")
    wid = c * 16 + s
    r0 = s * zrows
    pltpu.sync_copy(zeros_hbm, acc.at[pl.ds(r0, zrows)])
    plsc.subcore_barrier()
    base = wid * EPT

    def step(i, carry):
      off = pl.multiple_of(base + i * CH, 8)
      cs = pltpu.async_copy(src_hbm.at[pl.ds(off, CH)], sidx, semA)
      cd = pltpu.async_copy(dst_hbm.at[pl.ds(off, CH)], didx, semB)
      cs.wait()
      pltpu.async_copy(table_hbm.at[sidx], rows, sem).wait()
      cd.wait()
      pltpu.sync_copy(rows, acc.at[didx], add=True)
      return carry

    lax.fori_loop(0, EPT // CH, step, 0)
    plsc.subcore_barrier()
    pltpu.sync_copy(acc.at[pl.ds(r0, zrows)], out_hbm.at[c, pl.ds(r0, zrows)])

  return k


_segsum_w144 = _make_segsum(144)
_segsum_w64 = _make_segsum(64)


@functools.partial(
    pl.kernel,
    out_type=(
        jax.ShapeDtypeStruct((NW * TPAD,), jnp.int32),
        jax.ShapeDtypeStruct((NW * TPAD,), jnp.int32),
        jax.ShapeDtypeStruct((NW * NW,), jnp.int32),
        jax.ShapeDtypeStruct((NW * NW,), jnp.int32),
    ),
    mesh=_mesh(),
    compiler_params=pltpu.CompilerParams(needs_layout_passes=False),
    scratch_types=[
        pltpu.VMEM((EPT,), jnp.int32),
        pltpu.VMEM((EPT,), jnp.int32),
        pltpu.VMEM((EPT,), jnp.int32),
        pltpu.VMEM((EPT,), jnp.int32),
        pltpu.VMEM((TPAD,), jnp.int32),
        pltpu.VMEM((TPAD,), jnp.int32),
        pltpu.VMEM((32,), jnp.int32),
        pltpu.VMEM((32,), jnp.int32),
    ],
)
def _sc_bucket(src_hbm, dst_hbm, zeros_hbm, bsrc_o, bdst_o, offs_o, cnts_o,
               srcl, dstl, tsrc, tdst, bsrc_loc, bdst_loc, hist, cur):
  wid = _wid()
  pltpu.sync_copy(src_hbm.at[pl.ds(pl.multiple_of(wid * EPT, 8), EPT)], srcl)
  pltpu.sync_copy(dst_hbm.at[pl.ds(pl.multiple_of(wid * EPT, 8), EPT)], dstl)
  pltpu.sync_copy(zeros_hbm.at[pl.ds(0, TPAD)], bsrc_loc)
  pltpu.sync_copy(zeros_hbm.at[pl.ds(0, TPAD)], bdst_loc)
  zi = jnp.zeros((16,), jnp.int32)
  lanes = lax.iota(jnp.int32, 16)

  def _hist_offsets(align):
    h0 = hist[pl.ds(0, 16)]
    h1 = hist[pl.ds(16, 16)]
    if align:
      h0 = (h0 + 7) & (-8)
      h1 = (h1 + 7) & (-8)
    e0 = plsc.cumsum(h0) - h0
    e1 = plsc.cumsum(h1) - h1 + jnp.sum(h0)
    cur[pl.ds(0, 16)] = e0
    cur[pl.ds(16, 16)] = e1

  def _pass(in_s, in_d, out_s, out_d, keyfn, validfn, nvec, align,
            export_offs=False):
    hist[pl.ds(0, 16)] = zi
    hist[pl.ds(16, 16)] = zi

    def hstep(i, carry):
      d = in_d[pl.ds(i * 16, 16)]
      m = validfn(i, d)
      b = jnp.where(m, keyfn(d), 0)
      r, lastm = plsc.scan_count(b, mask=m)
      plsc.addupdate_scatter(hist, [b], r, mask=lastm)
      return carry

    lax.fori_loop(0, nvec, hstep, 0)
    total = jnp.sum(hist[pl.ds(0, 16)]) + jnp.sum(hist[pl.ds(16, 16)])
    _hist_offsets(align)
    if export_offs:
      pltpu.sync_copy(cur, offs_o.at[pl.ds(pl.multiple_of(wid * 32, 8), 32)])
      pltpu.sync_copy(hist, cnts_o.at[pl.ds(pl.multiple_of(wid * 32, 8), 32)])

    def sstep(i, carry):
      sl = pl.ds(i * 16, 16)
      d = in_d[sl]
      sv = in_s[sl]
      m = validfn(i, d)
      b = jnp.where(m, keyfn(d), 0)
      r, lastm = plsc.scan_count(b, mask=m)
      base = plsc.load_gather(cur, [b], mask=m)
      pos = base + r - 1
      plsc.store_scatter(cur, [b], pos + 1, mask=lastm)
      plsc.store_scatter(out_s, [pos], sv, mask=m)
      plsc.store_scatter(out_d, [pos], d, mask=m)
      return carry

    lax.fori_loop(0, nvec, sstep, 0)
    return total

  cnt = _pass(srcl, dstl, tsrc, tdst,
              lambda d: d & 31,
              lambda i, d: d < N,
              EPT // 16, False)
  nv = (cnt + 15) // 16
  _pass(tsrc, tdst, srcl, dstl,
        lambda d: (d % NPB) >> 5,
        lambda i, d: i * 16 + lanes < cnt,
        nv, False)
  _pass(srcl, dstl, bsrc_loc, bdst_loc,
        lambda d: d // NPB,
        lambda i, d: i * 16 + lanes < cnt,
        nv, True, export_offs=True)
  pltpu.sync_copy(bsrc_loc,
                  bsrc_o.at[pl.ds(pl.multiple_of(wid * TPAD, 8), TPAD)])
  pltpu.sync_copy(bdst_loc,
                  bdst_o.at[pl.ds(pl.multiple_of(wid * TPAD, 8), TPAD)])


MCH = 128


@functools.partial(
    pl.kernel,
    out_type=jax.ShapeDtypeStruct((NP, 128), jnp.float32),
    mesh=_mesh(),
    compiler_params=pltpu.CompilerParams(needs_layout_passes=False),
    scratch_types=[
        pltpu.VMEM((MCH,), jnp.int32),
        pltpu.VMEM((MCH,), jnp.int32),
        pltpu.VMEM((NW * NW,), jnp.int32),
        pltpu.VMEM((NW * NW,), jnp.int32),
        pltpu.VMEM((MCH, 128), jnp.float32),
        pltpu.VMEM((NPB + 8, 128), jnp.float32),
        pltpu.SemaphoreType.DMA,
        pltpu.SemaphoreType.DMA,
        pltpu.SemaphoreType.DMA,
    ],
)
def _sc_segmax(bsrc, bdst, offs_hbm, cnts_hbm, table_hbm, zacc_hbm, out_hbm,
               sidx, dstv, offs_v, cnts_v, msgs, acc, sem, semA, semB):
  u = _wid()
  pltpu.sync_copy(zacc_hbm, acc)
  pltpu.sync_copy(offs_hbm, offs_v)
  pltpu.sync_copy(cnts_hbm, cnts_v)
  lo = u * NPB
  lanes = lax.iota(jnp.int32, 16)
  uhi = (u // 16) * 16
  ulane = u - uhi
  zv = jnp.zeros((16,), jnp.float32)

  def _flush(dp, regs):
    for j in range(8):
      sl = pl.ds(j * 16, 16)
      acc[dp, sl] = jnp.maximum(acc[dp, sl], regs[j])

  def tloop(t, tcarry):
    ovec = offs_v[pl.ds(pl.multiple_of(t * 32 + uhi, 16), 16)]
    cvec = cnts_v[pl.ds(pl.multiple_of(t * 32 + uhi, 16), 16)]
    off = jnp.sum(jnp.where(lanes == ulane, ovec, 0))
    cnt = jnp.sum(jnp.where(lanes == ulane, cvec, 0))
    nch = (cnt + MCH - 1) // MCH

    def chunk(i, carry):
      stt = pl.multiple_of(t * TPAD + off + i * MCH, 8)
      cs = pltpu.async_copy(bsrc.at[pl.ds(stt, MCH)], sidx, semA)
      cd = pltpu.async_copy(bdst.at[pl.ds(stt, MCH)], dstv, semB)
      cs.wait()
      g = pltpu.async_copy(table_hbm.at[sidx], msgs, sem)
      cd.wait()
      g.wait()
      rem = cnt - i * MCH

      def grp(gi, gcarry):
        dp = gcarry[0]
        regs = list(gcarry[1:])
        dvec = dstv[pl.ds(gi * 16, 16)] - lo
        for l in range(16):
          e = gi * 16 + l
          d0 = jnp.sum(jnp.where(lanes == l, dvec, 0))
          d = jnp.where(e < rem, d0, NPB)
          changed = d != dp

          @pl.when(changed)
          def _():
            _flush(dp, regs)

          for j in range(8):
            m = msgs[e, pl.ds(j * 16, 16)]
            regs[j] = jnp.where(changed, m, jnp.maximum(regs[j], m))
          dp = d
        return (dp, *regs)

      return lax.fori_loop(0, MCH // 16, grp, carry)

    return lax.fori_loop(0, nch, chunk, tcarry)

  fin = lax.fori_loop(0, NW, tloop,
                      (jnp.int32(NPB), zv, zv, zv, zv, zv, zv, zv, zv))
  _flush(fin[0], list(fin[1:]))
  pltpu.sync_copy(acc.at[pl.ds(0, NPB)],
                  out_hbm.at[pl.ds(pl.multiple_of(lo, 8), NPB)])


@functools.partial(
    pl.kernel,
    out_type=(
        jax.ShapeDtypeStruct((NW, G, 64), jnp.float32),
        jax.ShapeDtypeStruct((NW, G, 64), jnp.float32),
        jax.ShapeDtypeStruct((NW, G, 16), jnp.float32),
    ),
    mesh=_mesh(),
    compiler_params=pltpu.CompilerParams(needs_layout_passes=False),
    scratch_types=[
        pltpu.VMEM((NPB, 64), jnp.float32),
        pltpu.VMEM((NPB,), jnp.int32),
        pltpu.VMEM((G, 64), jnp.float32),
        pltpu.VMEM((G, 64), jnp.float32),
        pltpu.VMEM((G, 16), jnp.float32),
    ],
)
def _sc_pool(h3_hbm, batch_hbm, z64_hbm, z16_hbm, ps_o, pm_o, pc_o,
             rowsb, batchv, acc_s, acc_m, acc_c):
  w = _wid()
  base = pl.multiple_of(w * NPB, 8)
  pltpu.sync_copy(h3_hbm.at[pl.ds(base, NPB)], rowsb)
  pltpu.sync_copy(batch_hbm.at[pl.ds(base, NPB)], batchv)
  pltpu.sync_copy(z64_hbm, acc_s)
  pltpu.sync_copy(z64_hbm, acc_m)
  pltpu.sync_copy(z16_hbm, acc_c)
  lanes = lax.iota(jnp.int32, 16)
  onev = jnp.where(lanes == 0, 1.0, 0.0).astype(jnp.float32)

  def grp(gi, carry):
    bvec = batchv[pl.ds(gi * 16, 16)]
    for l in range(16):
      r = gi * 16 + l

      @pl.when(w * NPB + r < N)
      def _():
        g = jnp.sum(jnp.where(lanes == l, bvec, 0))
        for j in range(4):
          sl = pl.ds(j * 16, 16)
          v = rowsb[r, sl]
          acc_s[g, sl] = acc_s[g, sl] + v
          acc_m[g, sl] = jnp.maximum(acc_m[g, sl], v)
        c16 = pl.ds(0, 16)
        acc_c[g, c16] = acc_c[g, c16] + onev

    return carry

  lax.fori_loop(0, NPB // 16, grp, 0)
  pltpu.sync_copy(acc_s, ps_o.at[w])
  pltpu.sync_copy(acc_m, pm_o.at[w])
  pltpu.sync_copy(acc_c, pc_o.at[w])


_RB = 2000


def _mm_body(x_ref, w_ref, o_ref):
  o_ref[...] = jnp.dot(x_ref[...], w_ref[...],
                       preferred_element_type=jnp.float32)


def _tc_mm(x, w):
  n, kdim = x.shape
  m = w.shape[1]
  return pl.pallas_call(
      _mm_body,
      grid=(n // _RB,),
      in_specs=[
          pl.BlockSpec((_RB, kdim), lambda i: (i, 0)),
          pl.BlockSpec((kdim, m), lambda i: (0, 0)),
      ],
      out_specs=pl.BlockSpec((_RB, m), lambda i: (i, 0)),
      out_shape=jax.ShapeDtypeStruct((n, m), jnp.float32),
  )(x, w)


def _layer_mean_body(s0_ref, s1_ref, d0_ref, d1_ref, x_ref, w_ref, b_ref,
                     o_ref):
  s = s0_ref[...] + s1_ref[...]
  deg = jnp.sum(d0_ref[...] + d1_ref[...], axis=1, keepdims=True)
  agg = s / jnp.maximum(deg, 1.0)
  o_ref[...] = jax.nn.relu(agg + b_ref[...] +
                           jnp.dot(x_ref[...], w_ref[...],
                                   preferred_element_type=jnp.float32))


def _tc_layer_mean(s0, s1, d0, d1, x, w, b):
  n, kdim = x.shape
  m = w.shape[1]
  return pl.pallas_call(
      _layer_mean_body,
      grid=(n // _RB,),
      in_specs=[
          pl.BlockSpec((_RB, m), lambda i: (i, 0)),
          pl.BlockSpec((_RB, m), lambda i: (i, 0)),
          pl.BlockSpec((_RB, 16), lambda i: (i, 0)),
          pl.BlockSpec((_RB, 16), lambda i: (i, 0)),
          pl.BlockSpec((_RB, kdim), lambda i: (i, 0)),
          pl.BlockSpec((kdim, m), lambda i: (0, 0)),
          pl.BlockSpec((1, m), lambda i: (0, 0)),
      ],
      out_specs=pl.BlockSpec((_RB, m), lambda i: (i, 0)),
      out_shape=jax.ShapeDtypeStruct((n, m), jnp.float32),
  )(s0, s1, d0, d1, x, w, b)


def _layer2_body(m_ref, h1_ref, wl_ref, wr_ref, b_ref, w3_ref, h2_ref,
                 p3_ref):
  h2 = jax.nn.relu(
      jnp.dot(m_ref[...], wl_ref[...], preferred_element_type=jnp.float32)
      + b_ref[...]
      + jnp.dot(h1_ref[...], wr_ref[...], preferred_element_type=jnp.float32))
  h2_ref[...] = h2
  p3_ref[...] = jnp.dot(h2, w3_ref[...], preferred_element_type=jnp.float32)


def _tc_layer2(m, h1, wl, wr, b, w3):
  n = m.shape[0]
  return pl.pallas_call(
      _layer2_body,
      grid=(n // _RB,),
      in_specs=[
          pl.BlockSpec((_RB, 128), lambda i: (i, 0)),
          pl.BlockSpec((_RB, 128), lambda i: (i, 0)),
          pl.BlockSpec((128, 128), lambda i: (0, 0)),
          pl.BlockSpec((128, 128), lambda i: (0, 0)),
          pl.BlockSpec((1, 128), lambda i: (0, 0)),
          pl.BlockSpec((128, 64), lambda i: (0, 0)),
      ],
      out_specs=(
          pl.BlockSpec((_RB, 128), lambda i: (i, 0)),
          pl.BlockSpec((_RB, 64), lambda i: (i, 0)),
      ),
      out_shape=(
          jax.ShapeDtypeStruct((n, 128), jnp.float32),
          jax.ShapeDtypeStruct((n, 64), jnp.float32),
      ),
  )(m, h1, wl, wr, b, w3)


def _gelu(z):
  return 0.5 * z * (1.0 + lax.erf(z * 0.7071067811865476))


def _lnorm(p, g, b):
  m = jnp.mean(p, axis=-1, keepdims=True)
  r = p - m
  v = jnp.mean(r * r, axis=-1, keepdims=True)
  return r * lax.rsqrt(v + 1e-5) * g + b


def _head_body(ps_ref, pm_ref, pc_ref, w1_ref, b1_ref, w2_ref, b2_ref,
               w3_ref, b3_ref, w4_ref, b4_ref, w5_ref, b5_ref, g1_ref,
               be1_ref, g2_ref, be2_ref, g3_ref, be3_ref, g4_ref, be4_ref,
               o_ref):
  ps = jnp.zeros((G, 64), jnp.float32)
  pm = jnp.zeros((G, 64), jnp.float32)
  pc = jnp.zeros((G, 16), jnp.float32)
  for t in range(NW):
    ps = ps + ps_ref[t]
    pm = jnp.maximum(pm, pm_ref[t])
    pc = pc + pc_ref[t]
  cnt = pc[:, 0:1]
  mean_p = ps / jnp.maximum(cnt, 1.0)
  p = jnp.concatenate([mean_p, pm, ps], axis=-1)
  p = _lnorm(_gelu(jnp.dot(p, w1_ref[...],
                           preferred_element_type=jnp.float32) + b1_ref[...]),
             g1_ref[...], be1_ref[...])
  p = _lnorm(_gelu(jnp.dot(p, w2_ref[...],
                           preferred_element_type=jnp.float32) + b2_ref[...]),
             g2_ref[...], be2_ref[...])
  p = _lnorm(_gelu(jnp.dot(p, w3_ref[...],
                           preferred_element_type=jnp.float32) + b3_ref[...]),
             g3_ref[...], be3_ref[...])
  p = _lnorm(_gelu(jnp.dot(p, w4_ref[...],
                           preferred_element_type=jnp.float32) + b4_ref[...]),
             g4_ref[...], be4_ref[...])
  o_ref[...] = jnp.dot(p, w5_ref[...],
                       preferred_element_type=jnp.float32) + b5_ref[...]


def _tc_head(ps, pm, pc, hw, hb, gs, bes):
  full = lambda s: pl.BlockSpec(s, lambda: tuple(0 for _ in s))
  in_specs = [full((NW, G, 64)), full((NW, G, 64)), full((NW, G, 16))]
  args = [ps, pm, pc]
  for i in range(5):
    in_specs += [full(hw[i].shape), full((1, hw[i].shape[1]))]
    args += [hw[i], hb[i].reshape(1, -1)]
  for i in range(4):
    in_specs += [full((1, gs[i].shape[0])), full((1, gs[i].shape[0]))]
    args += [gs[i].reshape(1, -1), bes[i].reshape(1, -1)]
  return pl.pallas_call(
      _head_body,
      in_specs=in_specs,
      out_specs=full((G, 9)),
      out_shape=jax.ShapeDtypeStruct((G, 9), jnp.float32),
  )(*args)


def kernel(x, edge_index, batch, W1l, b1, W1r, W2l, b2, W2r, W3l, b3, W3r,
           hW1, hb1, hW2, hb2, hW3, hb3, hW4, hb4, hW5, hb5, g1, be1, g2,
           be2, g3, be3, g4, be4):
  f32 = jnp.float32
  src = edge_index[0]
  dst = edge_index[1]
  npad = EPAD - E
  park = jnp.arange(npad, dtype=jnp.int32)
  src_p = jnp.concatenate([src, park % N])
  dst_p = jnp.concatenate([dst, N + park % (NP - N)])

  zeros_w144 = jnp.zeros((NP // 16, 144), f32)
  zeros_w64 = jnp.zeros((NP // 16, 64), f32)
  zeros_i32 = jnp.zeros((TPAD,), jnp.int32)
  zacc = jnp.zeros((NPB + 8, 128), f32)
  z64 = jnp.zeros((G, 64), f32)
  z16 = jnp.zeros((G, 16), f32)

  p1 = _tc_mm(x, W1l)
  p1aug = jnp.concatenate(
      [p1, jnp.ones((N, 1), f32), jnp.zeros((N, 15), f32)], axis=1)
  s1 = _segsum_w144(src_p, dst_p, p1aug, zeros_w144)
  s1a = s1[0, :, :128]
  s1b = s1[1, :, :128]
  d1a = s1[0, :, 128:144]
  d1b = s1[1, :, 128:144]
  h1 = _tc_layer_mean(s1a, s1b, d1a, d1b, x, W1r, b1.reshape(1, -1))

  bsrc, bdst, offs, cnts = _sc_bucket(src_p, dst_p, zeros_i32)

  m = _sc_segmax(bsrc, bdst, offs, cnts, h1, zacc)
  h2, p3 = _tc_layer2(m[:N], h1, W2l, W2r, b2.reshape(1, -1), W3l)

  s3 = _segsum_w64(src_p, dst_p, p3, zeros_w64)
  h3 = _tc_layer_mean(s3[0], s3[1], d1a, d1b, h2, W3r, b3.reshape(1, -1))

  h3p = jnp.concatenate([h3, jnp.zeros((NP - N, 64), f32)])
  batch_p = jnp.concatenate([batch, jnp.zeros((NP - N,), jnp.int32)])
  ps, pm, pc = _sc_pool(h3p, batch_p, z64, z16)
  return _tc_head(ps, pm, pc, (hW1, hW2, hW3, hW4, hW5),
                  (hb1, hb2, hb3, hb4, hb5), (g1, g2, g3, g4),
                  (be1, be2, be3, be4))
